# Initial kernel scaffold; baseline (speedup 1.0000x reference)
import jax
import jax.numpy as jnp
from jax.experimental import pallas as pl


def kernel(x, edge_index, edge_weight, W, b):
    raise NotImplementedError("write your pallas kernel here")


# speedup vs baseline: 6.5302x; 6.5302x over previous
import functools

import jax
import jax.numpy as jnp
from jax import lax
from jax.experimental import pallas as pl
from jax.experimental.pallas import tpu as pltpu
from jax.experimental.pallas import tpu_sc as plsc

_N = 10000
_E = 320000
_D = 128

_NC = 2
_NS = 16
_NW = _NC * _NS
_L = 16

_EPW = _E // _NW
_C = 80
_SC_E = 2000
_NSUPER = _EPW // _SC_E
_NCHUNK = _SC_E // _C
_DR = 80
_WB_STRIDE = 624
_WB_ROWS = 640


def _sc_aggregate(x, src_f, dst_f, ew_f):
  mesh = plsc.VectorSubcoreMesh(core_axis_name="c", subcore_axis_name="s")

  @functools.partial(
      pl.kernel,
      out_type=[
          jax.ShapeDtypeStruct((_NC, _N, _D), jnp.float32),
          jax.ShapeDtypeStruct((_NC, _DR, _D), jnp.float32),
      ],
      mesh=mesh,
      scratch_types=[
          pltpu.VMEM_SHARED((_N, _D), jnp.float32),
          pltpu.VMEM_SHARED((_DR, _D), jnp.float32),
          pltpu.VMEM((_SC_E,), jnp.int32),
          pltpu.VMEM((_SC_E,), jnp.int32),
          pltpu.VMEM((_SC_E,), jnp.float32),
          pltpu.VMEM((_C,), jnp.int32),
          pltpu.VMEM((_C, _D), jnp.float32),
          pltpu.VMEM((_DR, _D), jnp.float32),
          pltpu.VMEM((_DR,), jnp.int32),
          pltpu.SemaphoreType.DMA,
      ],
  )
  def agg(x_hbm, src_hbm, dst_hbm, ew_hbm, acc_out, deg_out,
          acc_sh, deg_sh, src_v, dst_v, ew_v, dst80_v, rows_v, hist_v,
          rowidx_v, sem):
    cid = lax.axis_index("c")
    sid = lax.axis_index("s")
    wid = sid * _NC + cid
    ebase = wid * _EPW

    zv = jnp.zeros((_L,), jnp.float32)

    def hzero(i, _):
      for j in range(_D // _L):
        hist_v[i, pl.ds(j * _L, _L)] = zv
      return 0
    lax.fori_loop(0, _DR, hzero, 0)

    for g in range(_DR // _L):
      rowidx_v[pl.ds(g * _L, _L)] = lax.iota(jnp.int32, _L) + g * _L

    for k in range(_WB_ROWS // _DR):
      off = sid * _WB_STRIDE + k * _DR
      pltpu.sync_copy(hist_v, acc_sh.at[pl.ds(off, _DR)])

    @pl.when(sid == 0)
    def _():
      pltpu.sync_copy(hist_v, deg_sh)

    plsc.subcore_barrier()

    def super_body(sc, _):
      pltpu.sync_copy(src_hbm.at[pl.ds(ebase + sc * _SC_E, _SC_E)], src_v)
      pltpu.sync_copy(dst_hbm.at[pl.ds(ebase + sc * _SC_E, _SC_E)], dst_v)
      pltpu.sync_copy(ew_hbm.at[pl.ds(ebase + sc * _SC_E, _SC_E)], ew_v)

      def chunk_body(c, _):
        eoff = c * _C
        pltpu.async_copy(
            x_hbm.at[src_v.at[pl.ds(eoff, _C)]], rows_v, sem).wait()

        def grp_body(g, _):
          goff = eoff + g * _L
          wv16 = jnp.abs(ew_v[pl.ds(goff, _L)])
          dv16 = dst_v[pl.ds(goff, _L)]
          dst80_v[pl.ds(g * _L, _L)] = dv16
          for l in range(_L):
            e = g * _L + l
            wv = jnp.full((_L,), wv16[l], jnp.float32)
            for j in range(_D // _L):
              sl = pl.ds(j * _L, _L)
              rows_v[e, sl] = rows_v[e, sl] * wv
            dn = dv16[l]
            oh = jnp.where(lax.iota(jnp.int32, _L) == lax.bitwise_and(dn, 15),
                           1.0, 0.0).astype(jnp.float32)
            plsc.addupdate(
                hist_v.at[lax.shift_right_logical(dn, 7),
                          pl.ds(lax.bitwise_and(dn, 112), _L)], oh)
          return 0
        lax.fori_loop(0, _C // _L, grp_body, 0)

        pltpu.sync_copy(rows_v, acc_sh.at[dst80_v], add=True)
        return 0
      lax.fori_loop(0, _NCHUNK, chunk_body, 0)
      return 0
    lax.fori_loop(0, _NSUPER, super_body, 0)

    pltpu.sync_copy(hist_v, deg_sh.at[rowidx_v], add=True)
    plsc.subcore_barrier()

    off = sid * _WB_STRIDE
    pltpu.sync_copy(acc_sh.at[pl.ds(off, _WB_ROWS)],
                    acc_out.at[cid, pl.ds(off, _WB_ROWS)])

    @pl.when(sid == 0)
    def _():
      pltpu.sync_copy(deg_sh, deg_out.at[cid])

  return agg(x, src_f, dst_f, ew_f)


_BR = 1000


def _tc_body(acc_ref, deg_ref, w_ref, b_ref, o_ref):
  s = acc_ref[0] + acc_ref[1]
  d = deg_ref[0] + deg_ref[1]
  y = s / jnp.maximum(d, 1.0)
  o_ref[...] = lax.dot_general(
      y, w_ref[...], (((1,), (1,)), ((), ())),
      preferred_element_type=jnp.float32) + b_ref[...]


def _tc_finish(acc, deg, W, b2):
  return pl.pallas_call(
      _tc_body,
      out_shape=jax.ShapeDtypeStruct((_N, _D), jnp.float32),
      grid=(_N // _BR,),
      in_specs=[
          pl.BlockSpec((_NC, _BR, _D), lambda i: (0, i, 0)),
          pl.BlockSpec((_NC, _BR, 1), lambda i: (0, i, 0)),
          pl.BlockSpec((_D, _D), lambda i: (0, 0)),
          pl.BlockSpec((1, _D), lambda i: (0, 0)),
      ],
      out_specs=pl.BlockSpec((_BR, _D), lambda i: (i, 0)),
  )(acc, deg, W, b2)


@jax.jit
def kernel(x, edge_index, edge_weight, W, b):
  src_f = edge_index[1].reshape(_E)
  dst_f = edge_index[0].reshape(_E)
  acc, deg_packed = _sc_aggregate(x, src_f, dst_f, edge_weight)
  deg = deg_packed.reshape(_NC, _DR * _D, 1)[:, :_N, :]
  return _tc_finish(acc, deg, W, b.reshape(1, _D))


# speedup vs baseline: 10.9263x; 1.6732x over previous
import functools

import jax
import jax.numpy as jnp
from jax import lax
from jax.experimental import pallas as pl
from jax.experimental.pallas import tpu as pltpu
from jax.experimental.pallas import tpu_sc as plsc

_N = 10000
_E = 320000
_D = 128

_NC = 2
_NS = 16
_NW = _NC * _NS
_L = 16

_EPW = _E // _NW
_C = 80
_SC_E = 2000
_NSUPER = _EPW // _SC_E
_NCHUNK = _SC_E // _C
_EB = 2 * _SC_E
_DR = 80
_WB_STRIDE = 624
_WB_ROWS = 640


def _sc_aggregate(x, edata, ew):
  mesh = plsc.VectorSubcoreMesh(core_axis_name="c", subcore_axis_name="s")

  @functools.partial(
      pl.kernel,
      out_type=[
          jax.ShapeDtypeStruct((_NC, _N, _D), jnp.float32),
          jax.ShapeDtypeStruct((_NC, _DR, _D), jnp.float32),
      ],
      mesh=mesh,
      scratch_types=[
          pltpu.VMEM_SHARED((_N, _D), jnp.float32),
          pltpu.VMEM_SHARED((_DR, _D), jnp.float32),
          pltpu.VMEM((_EB,), jnp.int32),
          pltpu.VMEM((_SC_E,), jnp.float32),
          pltpu.VMEM((_C, _D), jnp.float32),
          pltpu.VMEM((_C, _D), jnp.float32),
          pltpu.VMEM((_C, _D), jnp.float32),
          pltpu.VMEM((_C,), jnp.int32),
          pltpu.VMEM((_C,), jnp.int32),
          pltpu.VMEM((_C,), jnp.int32),
          pltpu.VMEM((_DR, _D), jnp.float32),
          pltpu.VMEM((_DR,), jnp.int32),
          pltpu.SemaphoreType.DMA,
          pltpu.SemaphoreType.DMA,
          pltpu.SemaphoreType.DMA,
          pltpu.SemaphoreType.DMA,
          pltpu.SemaphoreType.DMA,
          pltpu.SemaphoreType.DMA,
      ],
  )
  def agg(x_hbm, edata_hbm, ew_hbm, acc_out, deg_out,
          acc_sh, deg_sh, ebuf, ewbuf, rows0, rows1, rows2,
          d80_0, d80_1, d80_2, hist_v, rowidx_v,
          sg0, sg1, sg2, ss0, ss1, ss2):
    cid = lax.axis_index("c")
    sid = lax.axis_index("s")
    wid = sid * _NC + cid

    rows = (rows0, rows1, rows2)
    d80 = (d80_0, d80_1, d80_2)
    sg = (sg0, sg1, sg2)
    ss = (ss0, ss1, ss2)

    zv = jnp.zeros((_L,), jnp.float32)

    def hzero(i, _):
      for j in range(_D // _L):
        hist_v[i, pl.ds(j * _L, _L)] = zv
      return 0
    lax.fori_loop(0, _DR, hzero, 0)

    for g in range(_DR // _L):
      rowidx_v[pl.ds(g * _L, _L)] = lax.iota(jnp.int32, _L) + g * _L

    for k in range(_WB_ROWS // _DR):
      off = sid * _WB_STRIDE + k * _DR
      pltpu.sync_copy(hist_v, acc_sh.at[pl.ds(off, _DR)])

    @pl.when(sid == 0)
    def _():
      pltpu.sync_copy(hist_v, deg_sh)

    plsc.subcore_barrier()

    def issue_gather(w, b):
      pltpu.async_copy(
          x_hbm.at[ebuf.at[pl.ds(w * _C, _C)]], rows[b], sg[b])

    def chunk_op(w, b):
      base = w * _C
      rows_b = rows[b]
      d80_b = d80[b]
      prev = (b + 2) % 3

      pltpu.make_async_copy(
          x_hbm.at[ebuf.at[pl.ds(0, _C)]], rows_b, sg[b]).wait()

      def grp_body(g, _):
        goff = base + g * _L
        wv16 = jnp.abs(ewbuf[pl.ds(goff, _L)])
        dv16 = ebuf[pl.ds(_SC_E + goff, _L)]
        d80_b[pl.ds(g * _L, _L)] = dv16
        for l in range(_L):
          e = g * _L + l
          wv = jnp.full((_L,), wv16[l], jnp.float32)
          for j in range(_D // _L):
            sl = pl.ds(j * _L, _L)
            rows_b[e, sl] = rows_b[e, sl] * wv
          dn = dv16[l]
          oh = jnp.where(lax.iota(jnp.int32, _L) == lax.bitwise_and(dn, 15),
                         1.0, 0.0).astype(jnp.float32)
          plsc.addupdate(
              hist_v.at[lax.shift_right_logical(dn, 7),
                        pl.ds(lax.bitwise_and(dn, 112), _L)], oh)
        return 0
      lax.fori_loop(0, _C // _L, grp_body, 0)

      pltpu.async_copy(rows_b, acc_sh.at[d80_b], ss[b], add=True)

      @pl.when(w >= 1)
      def _():
        pltpu.make_async_copy(rows[prev], acc_sh.at[d80[prev]],
                              ss[prev]).wait()

      @pl.when(w < _NCHUNK - 2)
      def _():
        issue_gather(w + 2, prev)

    def super_body(s, _):
      blk = wid * _NSUPER + s
      pltpu.sync_copy(edata_hbm.at[pl.ds(blk * _EB, _EB)], ebuf)
      pltpu.sync_copy(ew_hbm.at[pl.ds(blk * _SC_E, _SC_E)], ewbuf)
      issue_gather(0, 0)
      issue_gather(1, 1)

      def inner(w, _):
        m = lax.rem(w, 3)
        for b in range(3):
          @pl.when(m == b)
          def _():
            chunk_op(w, b)
        return 0
      lax.fori_loop(0, _NCHUNK, inner, 0)

      lastb = (_NCHUNK - 1) % 3
      pltpu.make_async_copy(rows[lastb], acc_sh.at[d80[lastb]],
                            ss[lastb]).wait()
      return 0
    lax.fori_loop(0, _NSUPER, super_body, 0)

    pltpu.sync_copy(hist_v, deg_sh.at[rowidx_v], add=True)
    plsc.subcore_barrier()

    off = sid * _WB_STRIDE
    pltpu.sync_copy(acc_sh.at[pl.ds(off, _WB_ROWS)],
                    acc_out.at[cid, pl.ds(off, _WB_ROWS)])

    @pl.when(sid == 0)
    def _():
      pltpu.sync_copy(deg_sh, deg_out.at[cid])

  return agg(x, edata, ew)


_BR = 1000


def _tc_body(acc_ref, deg_ref, w_ref, b_ref, o_ref):
  s = acc_ref[0] + acc_ref[1]
  d = deg_ref[0] + deg_ref[1]
  y = s / jnp.maximum(d, 1.0)
  o_ref[...] = lax.dot_general(
      y, w_ref[...], (((1,), (1,)), ((), ())),
      preferred_element_type=jnp.float32) + b_ref[...]


def _tc_finish(acc, deg, W, b2):
  return pl.pallas_call(
      _tc_body,
      out_shape=jax.ShapeDtypeStruct((_N, _D), jnp.float32),
      grid=(_N // _BR,),
      in_specs=[
          pl.BlockSpec((_NC, _BR, _D), lambda i: (0, i, 0)),
          pl.BlockSpec((_NC, _BR, 1), lambda i: (0, i, 0)),
          pl.BlockSpec((_D, _D), lambda i: (0, 0)),
          pl.BlockSpec((1, _D), lambda i: (0, 0)),
      ],
      out_specs=pl.BlockSpec((_BR, _D), lambda i: (i, 0)),
  )(acc, deg, W, b2)


@jax.jit
def kernel(x, edge_index, edge_weight, W, b):
  srcr = edge_index[1].reshape(_NW, _NSUPER, 1, _SC_E)
  dstr = edge_index[0].reshape(_NW, _NSUPER, 1, _SC_E)
  edata = jnp.concatenate([srcr, dstr], axis=2).reshape(-1)
  acc, deg_packed = _sc_aggregate(x, edata, edge_weight)
  deg = deg_packed.reshape(_NC, _DR * _D, 1)[:, :_N, :]
  return _tc_finish(acc, deg, W, b.reshape(1, _D))
